# baseline (speedup 1.0000x reference)
import functools

import jax
import jax.numpy as jnp
from jax import lax
from jax.experimental import pallas as pl
from jax.experimental.pallas import tpu as pltpu
from jax.experimental.pallas import tpu_sc as plsc

_NC = 2
_NS = 16
_C = 128
_G = 2

_N = 10000
_D = 128
_DH = _D // _NC
_NACC = 10240
_ZROWS = _NACC // _NS

_BLK = 2000


def _mesh():
    return plsc.VectorSubcoreMesh(core_axis_name="c", subcore_axis_name="s")


def _make_sc_prop(kp):
    ngroups = kp // _G

    @functools.partial(
        pl.kernel,
        out_type=jax.ShapeDtypeStruct((_NC * _NACC, _DH), jnp.float32),
        mesh=_mesh(),
        scratch_types=[
            pltpu.VMEM((kp, _C), jnp.int32),
            pltpu.VMEM((kp, _C), jnp.int32),
            pltpu.VMEM((_G * _C, _DH), jnp.float32),
            pltpu.VMEM((_G * _C, _DH), jnp.float32),
            pltpu.VMEM_SHARED((_NACC, _DH), jnp.float32),
            pltpu.SemaphoreType.DMA,
            pltpu.SemaphoreType.DMA,
            pltpu.SemaphoreType.DMA,
            pltpu.SemaphoreType.DMA,
        ],
        compiler_params=pltpu.CompilerParams(use_tc_tiling_on_sc=False),
    )
    def sc_prop(tab_hbm, srcs_hbm, dsts_hbm, zeros_hbm, out_hbm,
                src_v, dst_v, rows_a, rows_b, acc_sh,
                sem_ga, sem_gb, sem_sa, sem_sb):
        cid = lax.axis_index("c")
        sid = lax.axis_index("s")

        pltpu.sync_copy(zeros_hbm.at[pl.ds(sid * _ZROWS, _ZROWS)],
                        acc_sh.at[pl.ds(sid * _ZROWS, _ZROWS)])
        pltpu.sync_copy(srcs_hbm.at[pl.ds((cid * _NS + sid) * kp, kp)], src_v)
        pltpu.sync_copy(dsts_hbm.at[pl.ds(sid * kp, kp)], dst_v)
        plsc.subcore_barrier()

        def fire_gather(grp, buf, sem):
            base = grp * _G
            for q in range(_G):
                pltpu.async_copy(tab_hbm.at[src_v.at[base + q]],
                                 buf.at[pl.ds(q * _C, _C)], sem)

        def drain(buf, sem):
            for q in range(_G):
                pltpu.make_async_copy(tab_hbm.at[src_v.at[q]],
                                      buf.at[pl.ds(q * _C, _C)], sem).wait()

        def fire_scatter(grp, buf, sem):
            base = grp * _G
            for q in range(_G):
                pltpu.async_copy(buf.at[pl.ds(q * _C, _C)],
                                 acc_sh.at[dst_v.at[base + q]], sem, add=True)

        def drain_scatter(buf, sem):
            for q in range(_G):
                pltpu.make_async_copy(buf.at[pl.ds(q * _C, _C)],
                                      acc_sh.at[dst_v.at[q]], sem).wait()

        fire_gather(0, rows_a, sem_ga)
        fire_gather(1, rows_b, sem_gb)

        def pipe(i, carry):
            t = 2 * i
            drain(rows_a, sem_ga)
            fire_scatter(t, rows_a, sem_sa)
            drain(rows_b, sem_gb)
            fire_scatter(t + 1, rows_b, sem_sb)
            drain_scatter(rows_a, sem_sa)
            fire_gather(lax.rem(t + 2, ngroups), rows_a, sem_ga)
            drain_scatter(rows_b, sem_sb)
            fire_gather(lax.rem(t + 3, ngroups), rows_b, sem_gb)
            return carry

        lax.fori_loop(0, ngroups // 2, pipe, 0)
        drain(rows_a, sem_ga)
        drain(rows_b, sem_gb)
        plsc.subcore_barrier()

        pltpu.sync_copy(
            acc_sh.at[pl.ds(sid * _ZROWS, _ZROWS)],
            out_hbm.at[pl.ds(cid * _NACC + sid * _ZROWS, _ZROWS)])

    return sc_prop


def _make_sc_deg(kd):
    @functools.partial(
        pl.kernel,
        out_type=jax.ShapeDtypeStruct((_NC * _NACC, 16), jnp.float32),
        mesh=_mesh(),
        scratch_types=[
            pltpu.VMEM((kd, _C), jnp.int32),
            pltpu.VMEM((_C, 16), jnp.float32),
            pltpu.VMEM_SHARED((_NACC, 16), jnp.float32),
        ],
        compiler_params=pltpu.CompilerParams(use_tc_tiling_on_sc=False),
    )
    def sc_deg(srcs_hbm, zeros_hbm, ones_hbm, out_hbm,
               src_v, ones_v, acc_sh):
        cid = lax.axis_index("c")
        sid = lax.axis_index("s")
        wid = sid * _NC + cid

        pltpu.sync_copy(zeros_hbm.at[pl.ds(sid * _ZROWS, _ZROWS)],
                        acc_sh.at[pl.ds(sid * _ZROWS, _ZROWS)])
        pltpu.sync_copy(ones_hbm, ones_v)
        pltpu.sync_copy(srcs_hbm.at[pl.ds(wid * kd, kd)], src_v)
        plsc.subcore_barrier()

        def chunk(j, carry):
            pltpu.sync_copy(ones_v, acc_sh.at[src_v.at[j]], add=True)
            return carry

        lax.fori_loop(0, kd, chunk, 0)
        plsc.subcore_barrier()

        pltpu.sync_copy(
            acc_sh.at[pl.ds(sid * _ZROWS, _ZROWS)],
            out_hbm.at[pl.ds(cid * _NACC + sid * _ZROWS, _ZROWS)])

    return sc_deg


def _sblocks(i):
    del i
    return [
        pl.BlockSpec((1, _BLK, _DH), lambda i: (0, i, 0)),
        pl.BlockSpec((1, _BLK, _DH), lambda i: (1, i, 0)),
    ]


def _prep_body(dp_ref, x_ref, dis_ref, g0_ref):
    dp = dp_ref[...]
    deg = dp[0] + dp[1]
    dis = jnp.where(deg > 0, lax.rsqrt(jnp.where(deg > 0, deg, 1.0)), 0.0)
    dis_ref[...] = dis
    x = x_ref[...]
    dis_c = dis[:, 0:1]
    g0_ref[...] = jnp.stack([dis_c * x[:, :_DH], dis_c * x[:, _DH:]])


def _tc_prep(deg_parts, x):
    grid = _N // _BLK
    return pl.pallas_call(
        _prep_body,
        grid=(grid,),
        in_specs=[
            pl.BlockSpec((_NC, _BLK, 16), lambda i: (0, i, 0)),
            pl.BlockSpec((_BLK, _D), lambda i: (i, 0)),
        ],
        out_specs=[
            pl.BlockSpec((_BLK, 16), lambda i: (i, 0)),
            pl.BlockSpec((_NC, _BLK, _DH), lambda i: (0, i, 0)),
        ],
        out_shape=[
            jax.ShapeDtypeStruct((_N, 16), jnp.float32),
            jax.ShapeDtypeStruct((_NC, _N, _DH), jnp.float32),
        ],
    )(deg_parts, x)


def _scaleg_body(s0_ref, s1_ref, dis_ref, g_ref):
    nd2 = -jnp.square(dis_ref[...][:, 0:1])
    g_ref[...] = jnp.stack([nd2 * s0_ref[0], nd2 * s1_ref[0]])


def _tc_scaleg(s3d, dis):
    grid = _N // _BLK
    return pl.pallas_call(
        _scaleg_body,
        grid=(grid,),
        in_specs=_sblocks(0) + [pl.BlockSpec((_BLK, 16), lambda i: (i, 0))],
        out_specs=pl.BlockSpec((_NC, _BLK, _DH), lambda i: (0, i, 0)),
        out_shape=jax.ShapeDtypeStruct((_NC, _N, _DH), jnp.float32),
    )(s3d, s3d, dis)


def _dense_body(relu, h_ref, sa0_ref, sa1_ref, sb0_ref, sb1_ref,
                w_ref, b_ref, dis_ref, out_ref, g_ref):
    h = h_ref[...]
    nd = -dis_ref[...][:, 0:1]
    p1 = jnp.concatenate([nd * sa0_ref[0], nd * sa1_ref[0]], axis=1)
    p2 = jnp.concatenate([nd * sb0_ref[0], nd * sb1_ref[0]], axis=1)
    acc = jnp.dot(h, w_ref[0], preferred_element_type=jnp.float32)
    acc += jnp.dot(p1, w_ref[1], preferred_element_type=jnp.float32)
    acc += jnp.dot(2.0 * p2 - h, w_ref[2],
                   preferred_element_type=jnp.float32)
    acc += b_ref[...]
    if relu:
        acc = jnp.maximum(acc, 0.0)
    out_ref[...] = acc
    dis_c = dis_ref[...][:, 0:1]
    g_ref[...] = jnp.stack([dis_c * acc[:, :_DH], dis_c * acc[:, _DH:]])


def _tc_dense(h, sa3d, sb3d, w, b, dis, relu):
    grid = _N // _BLK
    return pl.pallas_call(
        functools.partial(_dense_body, relu),
        grid=(grid,),
        in_specs=(
            [pl.BlockSpec((_BLK, _D), lambda i: (i, 0))]
            + _sblocks(0) + _sblocks(0)
            + [
                pl.BlockSpec((3, _D, _D), lambda i: (0, 0, 0)),
                pl.BlockSpec((1, _D), lambda i: (0, 0)),
                pl.BlockSpec((_BLK, 16), lambda i: (i, 0)),
            ]
        ),
        out_specs=[
            pl.BlockSpec((_BLK, _D), lambda i: (i, 0)),
            pl.BlockSpec((_NC, _BLK, _DH), lambda i: (0, i, 0)),
        ],
        out_shape=[
            jax.ShapeDtypeStruct((_N, _D), jnp.float32),
            jax.ShapeDtypeStruct((_NC, _N, _DH), jnp.float32),
        ],
    )(h, sa3d, sa3d, sb3d, sb3d, w, b.reshape(1, _D), dis)


def kernel(x, edge, w1, b1, w2, b2):
    n, d = x.shape
    e = edge.shape[1]
    src = edge[0].astype(jnp.int32)
    dst = edge[1].astype(jnp.int32)

    kd = (-(-e // (_NC * _NS * _C)) + 7) // 8 * 8
    pad_d = _NC * _NS * kd * _C - e
    src_deg = jnp.concatenate(
        [src, jnp.full((pad_d,), n, jnp.int32)]).reshape(_NC * _NS * kd, _C)

    kp = (-(-e // (_NS * _C)) + 7) // 8 * 8
    pad_p = _NS * kp * _C - e
    src_p = jnp.concatenate([src, jnp.zeros((pad_p,), jnp.int32)])
    src_fs = jnp.concatenate(
        [src_p, src_p + jnp.int32(n)]).reshape(_NC * _NS * kp, _C)
    dst_fs = jnp.concatenate(
        [dst, jnp.full((pad_p,), n, jnp.int32)]).reshape(_NS * kp, _C)

    zeros_h = jnp.zeros((_NACC, _DH), jnp.float32)
    zeros16 = jnp.zeros((_NACC, 16), jnp.float32)
    ones16 = jnp.ones((_C, 16), jnp.float32)

    sc_deg = _make_sc_deg(kd)
    sc_prop = _make_sc_prop(kp)

    deg_parts = sc_deg(src_deg, zeros16, ones16).reshape(_NC, _NACC, 16)
    dis, g0 = _tc_prep(deg_parts, x)

    def prop3d(g):
        s = sc_prop(g.reshape(_NC * n, _DH), src_fs, dst_fs, zeros_h)
        return s.reshape(_NC, _NACC, _DH)

    s1 = prop3d(g0)
    g1 = _tc_scaleg(s1, dis)
    s2 = prop3d(g1)
    out1, g2 = _tc_dense(x, s1, s2, w1, b1, dis, relu=True)
    s3 = prop3d(g2)
    g3 = _tc_scaleg(s3, dis)
    s4 = prop3d(g3)
    out, _ = _tc_dense(out1, s3, s4, w2, b2, dis, relu=False)
    return out


# speedup vs baseline: 1.1500x; 1.1500x over previous
import functools

import jax
import numpy as np
import jax.numpy as jnp
from jax import lax
from jax.experimental import pallas as pl
from jax.experimental.pallas import tpu as pltpu
from jax.experimental.pallas import tpu_sc as plsc

_NC = 2
_NS = 16
_C = 128
_G = 2

_N = 10000
_D = 128
_DH = _D // _NC
_NACC = 10240
_ZROWS = _NACC // _NS

_BLK = 2000


def _mesh():
    return plsc.VectorSubcoreMesh(core_axis_name="c", subcore_axis_name="s")


def _make_sc_prop(kp):
    ngroups = kp // _G

    @functools.partial(
        pl.kernel,
        out_type=jax.ShapeDtypeStruct((_NC * _NACC, _DH), jnp.float32),
        mesh=_mesh(),
        scratch_types=[
            pltpu.VMEM((kp, _C), jnp.int32),
            pltpu.VMEM((kp, _C), jnp.int32),
            pltpu.VMEM((_G * _C, _DH), jnp.bfloat16),
            pltpu.VMEM((_G * _C, _DH), jnp.bfloat16),
            pltpu.VMEM((_G * _C, _DH), jnp.float32),
            pltpu.VMEM_SHARED((_NACC, _DH), jnp.float32),
            pltpu.SemaphoreType.DMA,
            pltpu.SemaphoreType.DMA,
        ],
        compiler_params=pltpu.CompilerParams(
            use_tc_tiling_on_sc=False, needs_layout_passes=False),
    )
    def sc_prop(tab_hbm, srcs_hbm, dsts_hbm, zeros_hbm, out_hbm,
                src_v, dst_v, rows_a, rows_b, st_v, acc_sh,
                sem_ga, sem_gb):
        cid = lax.axis_index("c")
        sid = lax.axis_index("s")

        pltpu.sync_copy(zeros_hbm.at[pl.ds(sid * _ZROWS, _ZROWS)],
                        acc_sh.at[pl.ds(sid * _ZROWS, _ZROWS)])
        pltpu.sync_copy(srcs_hbm.at[pl.ds((cid * _NS + sid) * kp, kp)], src_v)
        pltpu.sync_copy(dsts_hbm.at[pl.ds(sid * kp, kp)], dst_v)
        plsc.subcore_barrier()

        def fire_gather(grp, buf, sem):
            base = grp * _G
            for q in range(_G):
                pltpu.async_copy(tab_hbm.at[src_v.at[base + q]],
                                 buf.at[pl.ds(q * _C, _C)], sem)

        def drain(buf, sem):
            for q in range(_G):
                pltpu.make_async_copy(tab_hbm.at[src_v.at[q]],
                                      buf.at[pl.ds(q * _C, _C)], sem).wait()

        def convert(buf):
            def body(r, carry):
                for u in range(_DH // 32):
                    v = buf[r, pl.ds(32 * u, 32)]
                    a, b = plsc.unpack(v, format=plsc.PackFormat.INTERLEAVED)
                    st_v[r, pl.ds(32 * u, 16)] = a
                    st_v[r, pl.ds(32 * u + 16, 16)] = b
                return carry
            lax.fori_loop(0, _G * _C, body, 0)

        def phase(grp, buf, sem, nxt):
            drain(buf, sem)
            convert(buf)
            for q in range(_G):
                pltpu.sync_copy(st_v.at[pl.ds(q * _C, _C)],
                                acc_sh.at[dst_v.at[grp * _G + q]], add=True)
            fire_gather(nxt, buf, sem)

        fire_gather(0, rows_a, sem_ga)
        fire_gather(1, rows_b, sem_gb)

        def pipe(i, carry):
            t = 2 * i
            phase(t, rows_a, sem_ga, lax.rem(t + 2, ngroups))
            phase(t + 1, rows_b, sem_gb, lax.rem(t + 3, ngroups))
            return carry

        lax.fori_loop(0, ngroups // 2, pipe, 0)
        drain(rows_a, sem_ga)
        drain(rows_b, sem_gb)
        plsc.subcore_barrier()

        pltpu.sync_copy(
            acc_sh.at[pl.ds(sid * _ZROWS, _ZROWS)],
            out_hbm.at[pl.ds(cid * _NACC + sid * _ZROWS, _ZROWS)])

    return sc_prop


def _make_sc_deg(kd):
    @functools.partial(
        pl.kernel,
        out_type=jax.ShapeDtypeStruct((_NC * _NACC, 16), jnp.float32),
        mesh=_mesh(),
        scratch_types=[
            pltpu.VMEM((kd, _C), jnp.int32),
            pltpu.VMEM((_C, 16), jnp.float32),
            pltpu.VMEM_SHARED((_NACC, 16), jnp.float32),
        ],
        compiler_params=pltpu.CompilerParams(
            use_tc_tiling_on_sc=False, needs_layout_passes=False),
    )
    def sc_deg(srcs_hbm, zeros_hbm, ones_hbm, out_hbm,
               src_v, ones_v, acc_sh):
        cid = lax.axis_index("c")
        sid = lax.axis_index("s")
        wid = sid * _NC + cid

        pltpu.sync_copy(zeros_hbm.at[pl.ds(sid * _ZROWS, _ZROWS)],
                        acc_sh.at[pl.ds(sid * _ZROWS, _ZROWS)])
        pltpu.sync_copy(ones_hbm, ones_v)
        pltpu.sync_copy(srcs_hbm.at[pl.ds(wid * kd, kd)], src_v)
        plsc.subcore_barrier()

        def chunk(j, carry):
            pltpu.sync_copy(ones_v, acc_sh.at[src_v.at[j]], add=True)
            return carry

        lax.fori_loop(0, kd, chunk, 0)
        plsc.subcore_barrier()

        pltpu.sync_copy(
            acc_sh.at[pl.ds(sid * _ZROWS, _ZROWS)],
            out_hbm.at[pl.ds(cid * _NACC + sid * _ZROWS, _ZROWS)])

    return sc_deg


def _sblocks(i):
    del i
    return [
        pl.BlockSpec((1, _BLK, _DH), lambda i: (0, i, 0)),
        pl.BlockSpec((1, _BLK, _DH), lambda i: (1, i, 0)),
    ]


def _prep_body(dp_ref, x_ref, m_ref, dis_ref, g0_ref):
    dp = dp_ref[...]
    deg = dp[0] + dp[1]
    dis = jnp.where(deg > 0, lax.rsqrt(jnp.where(deg > 0, deg, 1.0)), 0.0)
    dis_ref[...] = dis
    dis_c = dis[:, 0:1]
    t = jnp.dot(dis_c * x_ref[...], m_ref[...],
                preferred_element_type=jnp.float32).astype(jnp.bfloat16)
    g0_ref[...] = jnp.stack([t[:, :_DH], t[:, _DH:]])


def _tc_prep(deg_parts, x, m):
    grid = _N // _BLK
    return pl.pallas_call(
        _prep_body,
        grid=(grid,),
        in_specs=[
            pl.BlockSpec((_NC, _BLK, 16), lambda i: (0, i, 0)),
            pl.BlockSpec((_BLK, _D), lambda i: (i, 0)),
            pl.BlockSpec((_D, _D), lambda i: (0, 0)),
        ],
        out_specs=[
            pl.BlockSpec((_BLK, 16), lambda i: (i, 0)),
            pl.BlockSpec((_NC, _BLK, _DH), lambda i: (0, i, 0)),
        ],
        out_shape=[
            jax.ShapeDtypeStruct((_N, 16), jnp.float32),
            jax.ShapeDtypeStruct((_NC, _N, _DH), jnp.bfloat16),
        ],
    )(deg_parts, x, m)


def _scaleg_body(s0_ref, s1_ref, dis_ref, m_ref, g_ref):
    nd2 = -jnp.square(dis_ref[...][:, 0:1])
    full = jnp.concatenate([nd2 * s0_ref[0], nd2 * s1_ref[0]], axis=1)
    t = jnp.dot(full, m_ref[...],
                preferred_element_type=jnp.float32).astype(jnp.bfloat16)
    g_ref[...] = jnp.stack([t[:, :_DH], t[:, _DH:]])


def _tc_scaleg(s3d, dis, m):
    grid = _N // _BLK
    return pl.pallas_call(
        _scaleg_body,
        grid=(grid,),
        in_specs=_sblocks(0) + [
            pl.BlockSpec((_BLK, 16), lambda i: (i, 0)),
            pl.BlockSpec((_D, _D), lambda i: (0, 0)),
        ],
        out_specs=pl.BlockSpec((_NC, _BLK, _DH), lambda i: (0, i, 0)),
        out_shape=jax.ShapeDtypeStruct((_NC, _N, _DH), jnp.bfloat16),
    )(s3d, s3d, dis, m)


def _dense_body(relu, h_ref, sa0_ref, sa1_ref, sb0_ref, sb1_ref,
                w_ref, b_ref, dis_ref, m_ref, out_ref, g_ref):
    h = h_ref[...]
    nd = -dis_ref[...][:, 0:1]
    p1 = jnp.concatenate([nd * sa0_ref[0], nd * sa1_ref[0]], axis=1)
    p2 = jnp.concatenate([nd * sb0_ref[0], nd * sb1_ref[0]], axis=1)
    acc = jnp.dot(h, w_ref[0], preferred_element_type=jnp.float32)
    acc += jnp.dot(p1, w_ref[1], preferred_element_type=jnp.float32)
    acc += jnp.dot(2.0 * p2 - h, w_ref[2],
                   preferred_element_type=jnp.float32)
    acc += b_ref[...]
    if relu:
        acc = jnp.maximum(acc, 0.0)
    out_ref[...] = acc
    dis_c = dis_ref[...][:, 0:1]
    t = jnp.dot(dis_c * acc, m_ref[...],
                preferred_element_type=jnp.float32).astype(jnp.bfloat16)
    g_ref[...] = jnp.stack([t[:, :_DH], t[:, _DH:]])


def _tc_dense(h, sa3d, sb3d, w, b, dis, m, relu):
    grid = _N // _BLK
    return pl.pallas_call(
        functools.partial(_dense_body, relu),
        grid=(grid,),
        in_specs=(
            [pl.BlockSpec((_BLK, _D), lambda i: (i, 0))]
            + _sblocks(0) + _sblocks(0)
            + [
                pl.BlockSpec((3, _D, _D), lambda i: (0, 0, 0)),
                pl.BlockSpec((1, _D), lambda i: (0, 0)),
                pl.BlockSpec((_BLK, 16), lambda i: (i, 0)),
                pl.BlockSpec((_D, _D), lambda i: (0, 0)),
            ]
        ),
        out_specs=[
            pl.BlockSpec((_BLK, _D), lambda i: (i, 0)),
            pl.BlockSpec((_NC, _BLK, _DH), lambda i: (0, i, 0)),
        ],
        out_shape=[
            jax.ShapeDtypeStruct((_N, _D), jnp.float32),
            jax.ShapeDtypeStruct((_NC, _N, _DH), jnp.bfloat16),
        ],
    )(h, sa3d, sa3d, sb3d, sb3d, w, b.reshape(1, _D), dis, m)


def kernel(x, edge, w1, b1, w2, b2):
    n, d = x.shape
    e = edge.shape[1]
    src = edge[0].astype(jnp.int32)
    dst = edge[1].astype(jnp.int32)

    kd = (-(-e // (_NC * _NS * _C)) + 7) // 8 * 8
    pad_d = _NC * _NS * kd * _C - e
    src_deg = jnp.concatenate(
        [src, jnp.full((pad_d,), n, jnp.int32)]).reshape(_NC * _NS * kd, _C)

    kp = (-(-e // (_NS * _C)) + 7) // 8 * 8
    pad_p = _NS * kp * _C - e
    src_p = jnp.concatenate([src, jnp.zeros((pad_p,), jnp.int32)])
    src_fs = jnp.concatenate(
        [src_p, src_p + jnp.int32(n)]).reshape(_NC * _NS * kp, _C)
    dst_fs = jnp.concatenate(
        [dst, jnp.full((pad_p,), n, jnp.int32)]).reshape(_NS * kp, _C)

    zeros_h = jnp.zeros((_NACC, _DH), jnp.float32)
    zeros16 = jnp.zeros((_NACC, 16), jnp.float32)
    ones16 = jnp.ones((_C, 16), jnp.float32)

    sc_deg = _make_sc_deg(kd)
    sc_prop = _make_sc_prop(kp)

    t64 = np.empty((64,), np.int32)
    for u in range(2):
        for k in range(16):
            t64[32 * u + k] = 32 * u + 2 * k
            t64[32 * u + 16 + k] = 32 * u + 2 * k + 1
    t128 = np.concatenate([t64, t64 + 64])
    perm = np.argsort(t128)
    m_np = np.zeros((_D, _D), np.float32)
    m_np[perm, np.arange(_D)] = 1.0
    m = jnp.asarray(m_np)

    deg_parts = sc_deg(src_deg, zeros16, ones16).reshape(_NC, _NACC, 16)
    dis, g0 = _tc_prep(deg_parts, x, m)

    def prop3d(g):
        s = sc_prop(g.reshape(_NC * n, _DH), src_fs, dst_fs, zeros_h)
        return s.reshape(_NC, _NACC, _DH)

    s1 = prop3d(g0)
    g1 = _tc_scaleg(s1, dis, m)
    s2 = prop3d(g1)
    out1, g2 = _tc_dense(x, s1, s2, w1, b1, dis, m, relu=True)
    s3 = prop3d(g2)
    g3 = _tc_scaleg(s3, dis, m)
    s4 = prop3d(g3)
    out, _ = _tc_dense(out1, s3, s4, w2, b2, dis, m, relu=False)
    return out


# speedup vs baseline: 1.2814x; 1.1142x over previous
import functools

import jax
import numpy as np
import jax.numpy as jnp
from jax import lax
from jax.experimental import pallas as pl
from jax.experimental.pallas import tpu as pltpu
from jax.experimental.pallas import tpu_sc as plsc

_NC = 2
_NS = 16
_C = 128

_N = 10000
_D = 128
_DH = _D // _NC
_NACC = 10240
_ZROWS = _NACC // _NS

_BLK = 2000


def _mesh():
    return plsc.VectorSubcoreMesh(core_axis_name="c", subcore_axis_name="s")


def _make_sc_prop(kp):
    @functools.partial(
        pl.kernel,
        out_type=jax.ShapeDtypeStruct((_NC * _NACC, _DH), jnp.float32),
        mesh=_mesh(),
        scratch_types=[
            pltpu.VMEM((kp, _C), jnp.int32),
            pltpu.VMEM((kp, _C), jnp.int32),
            pltpu.VMEM((_C, _DH), jnp.bfloat16),
            pltpu.VMEM((_C, _DH), jnp.bfloat16),
            pltpu.VMEM((_C, _DH), jnp.float32),
            pltpu.VMEM((_C, _DH), jnp.float32),
            pltpu.VMEM_SHARED((_NACC, _DH), jnp.float32),
            pltpu.SemaphoreType.DMA,
            pltpu.SemaphoreType.DMA,
            pltpu.SemaphoreType.DMA,
            pltpu.SemaphoreType.DMA,
        ],
        compiler_params=pltpu.CompilerParams(
            use_tc_tiling_on_sc=False, needs_layout_passes=False),
    )
    def sc_prop(tab_hbm, srcs_hbm, dsts_hbm, zeros_hbm, out_hbm,
                src_v, dst_v, rows_a, rows_b, st_a, st_b, acc_sh,
                sem_ga, sem_gb, sem_sa, sem_sb):
        cid = lax.axis_index("c")
        sid = lax.axis_index("s")

        pltpu.sync_copy(zeros_hbm.at[pl.ds(sid * _ZROWS, _ZROWS)],
                        acc_sh.at[pl.ds(sid * _ZROWS, _ZROWS)])
        pltpu.sync_copy(srcs_hbm.at[pl.ds((cid * _NS + sid) * kp, kp)], src_v)
        pltpu.sync_copy(dsts_hbm.at[pl.ds(sid * kp, kp)], dst_v)
        plsc.subcore_barrier()

        def fire_gather(grp, buf, sem):
            pltpu.async_copy(tab_hbm.at[src_v.at[grp]], buf, sem)

        def drain_gather(buf, sem):
            pltpu.make_async_copy(tab_hbm.at[src_v.at[0]], buf, sem).wait()

        def fire_scatter(grp, stb, sem):
            pltpu.async_copy(stb, acc_sh.at[dst_v.at[grp]], sem, add=True)

        def drain_scatter(stb, sem):
            pltpu.make_async_copy(stb, acc_sh.at[dst_v.at[0]], sem).wait()

        def convert(buf, stb):
            def body(r, carry):
                for u in range(_DH // 32):
                    v = buf[r, pl.ds(32 * u, 32)]
                    a, b = plsc.unpack(v, format=plsc.PackFormat.INTERLEAVED)
                    stb[r, pl.ds(32 * u, 16)] = a
                    stb[r, pl.ds(32 * u + 16, 16)] = b
                return carry
            lax.fori_loop(0, _C, body, 0, unroll=4)

        fire_gather(0, rows_a, sem_ga)
        fire_gather(1, rows_b, sem_gb)

        drain_gather(rows_a, sem_ga)
        convert(rows_a, st_a)
        fire_scatter(0, st_a, sem_sa)
        fire_gather(2, rows_a, sem_ga)
        drain_gather(rows_b, sem_gb)
        convert(rows_b, st_b)
        fire_scatter(1, st_b, sem_sb)
        fire_gather(3, rows_b, sem_gb)

        def pipe(i, carry):
            t = 2 * i
            drain_gather(rows_a, sem_ga)
            drain_scatter(st_a, sem_sa)
            convert(rows_a, st_a)
            fire_scatter(t, st_a, sem_sa)
            fire_gather(lax.rem(t + 2, kp), rows_a, sem_ga)
            drain_gather(rows_b, sem_gb)
            drain_scatter(st_b, sem_sb)
            convert(rows_b, st_b)
            fire_scatter(t + 1, st_b, sem_sb)
            fire_gather(lax.rem(t + 3, kp), rows_b, sem_gb)
            return carry

        lax.fori_loop(1, kp // 2, pipe, 0)
        drain_scatter(st_a, sem_sa)
        drain_scatter(st_b, sem_sb)
        drain_gather(rows_a, sem_ga)
        drain_gather(rows_b, sem_gb)
        plsc.subcore_barrier()

        pltpu.sync_copy(
            acc_sh.at[pl.ds(sid * _ZROWS, _ZROWS)],
            out_hbm.at[pl.ds(cid * _NACC + sid * _ZROWS, _ZROWS)])

    return sc_prop


def _make_sc_deg(kd):
    @functools.partial(
        pl.kernel,
        out_type=jax.ShapeDtypeStruct((_NC * _NACC, 16), jnp.float32),
        mesh=_mesh(),
        scratch_types=[
            pltpu.VMEM((kd, _C), jnp.int32),
            pltpu.VMEM((_C, 16), jnp.float32),
            pltpu.VMEM_SHARED((_NACC, 16), jnp.float32),
        ],
        compiler_params=pltpu.CompilerParams(
            use_tc_tiling_on_sc=False, needs_layout_passes=False),
    )
    def sc_deg(srcs_hbm, zeros_hbm, ones_hbm, out_hbm,
               src_v, ones_v, acc_sh):
        cid = lax.axis_index("c")
        sid = lax.axis_index("s")
        wid = sid * _NC + cid

        pltpu.sync_copy(zeros_hbm.at[pl.ds(sid * _ZROWS, _ZROWS)],
                        acc_sh.at[pl.ds(sid * _ZROWS, _ZROWS)])
        pltpu.sync_copy(ones_hbm, ones_v)
        pltpu.sync_copy(srcs_hbm.at[pl.ds(wid * kd, kd)], src_v)
        plsc.subcore_barrier()

        def chunk(j, carry):
            pltpu.sync_copy(ones_v, acc_sh.at[src_v.at[j]], add=True)
            return carry

        lax.fori_loop(0, kd, chunk, 0)
        plsc.subcore_barrier()

        pltpu.sync_copy(
            acc_sh.at[pl.ds(sid * _ZROWS, _ZROWS)],
            out_hbm.at[pl.ds(cid * _NACC + sid * _ZROWS, _ZROWS)])

    return sc_deg


def _sblocks(i):
    del i
    return [
        pl.BlockSpec((1, _BLK, _DH), lambda i: (0, i, 0)),
        pl.BlockSpec((1, _BLK, _DH), lambda i: (1, i, 0)),
    ]


def _prep_body(dp_ref, x_ref, m_ref, dis_ref, g0_ref):
    dp = dp_ref[...]
    deg = dp[0] + dp[1]
    dis = jnp.where(deg > 0, lax.rsqrt(jnp.where(deg > 0, deg, 1.0)), 0.0)
    dis_ref[...] = dis
    dis_c = dis[:, 0:1]
    t = jnp.dot(dis_c * x_ref[...], m_ref[...],
                preferred_element_type=jnp.float32).astype(jnp.bfloat16)
    g0_ref[...] = jnp.stack([t[:, :_DH], t[:, _DH:]])


def _tc_prep(deg_parts, x, m):
    grid = _N // _BLK
    return pl.pallas_call(
        _prep_body,
        grid=(grid,),
        in_specs=[
            pl.BlockSpec((_NC, _BLK, 16), lambda i: (0, i, 0)),
            pl.BlockSpec((_BLK, _D), lambda i: (i, 0)),
            pl.BlockSpec((_D, _D), lambda i: (0, 0)),
        ],
        out_specs=[
            pl.BlockSpec((_BLK, 16), lambda i: (i, 0)),
            pl.BlockSpec((_NC, _BLK, _DH), lambda i: (0, i, 0)),
        ],
        out_shape=[
            jax.ShapeDtypeStruct((_N, 16), jnp.float32),
            jax.ShapeDtypeStruct((_NC, _N, _DH), jnp.bfloat16),
        ],
    )(deg_parts, x, m)


def _scaleg_body(s0_ref, s1_ref, dis_ref, m_ref, g_ref):
    nd2 = -jnp.square(dis_ref[...][:, 0:1])
    full = jnp.concatenate([nd2 * s0_ref[0], nd2 * s1_ref[0]], axis=1)
    t = jnp.dot(full, m_ref[...],
                preferred_element_type=jnp.float32).astype(jnp.bfloat16)
    g_ref[...] = jnp.stack([t[:, :_DH], t[:, _DH:]])


def _tc_scaleg(s3d, dis, m):
    grid = _N // _BLK
    return pl.pallas_call(
        _scaleg_body,
        grid=(grid,),
        in_specs=_sblocks(0) + [
            pl.BlockSpec((_BLK, 16), lambda i: (i, 0)),
            pl.BlockSpec((_D, _D), lambda i: (0, 0)),
        ],
        out_specs=pl.BlockSpec((_NC, _BLK, _DH), lambda i: (0, i, 0)),
        out_shape=jax.ShapeDtypeStruct((_NC, _N, _DH), jnp.bfloat16),
    )(s3d, s3d, dis, m)


def _dense_body(relu, h_ref, sa0_ref, sa1_ref, sb0_ref, sb1_ref,
                w_ref, b_ref, dis_ref, m_ref, out_ref, g_ref):
    h = h_ref[...]
    nd = -dis_ref[...][:, 0:1]
    p1 = jnp.concatenate([nd * sa0_ref[0], nd * sa1_ref[0]], axis=1)
    p2 = jnp.concatenate([nd * sb0_ref[0], nd * sb1_ref[0]], axis=1)
    acc = jnp.dot(h, w_ref[0], preferred_element_type=jnp.float32)
    acc += jnp.dot(p1, w_ref[1], preferred_element_type=jnp.float32)
    acc += jnp.dot(2.0 * p2 - h, w_ref[2],
                   preferred_element_type=jnp.float32)
    acc += b_ref[...]
    if relu:
        acc = jnp.maximum(acc, 0.0)
    out_ref[...] = acc
    dis_c = dis_ref[...][:, 0:1]
    t = jnp.dot(dis_c * acc, m_ref[...],
                preferred_element_type=jnp.float32).astype(jnp.bfloat16)
    g_ref[...] = jnp.stack([t[:, :_DH], t[:, _DH:]])


def _tc_dense(h, sa3d, sb3d, w, b, dis, m, relu):
    grid = _N // _BLK
    return pl.pallas_call(
        functools.partial(_dense_body, relu),
        grid=(grid,),
        in_specs=(
            [pl.BlockSpec((_BLK, _D), lambda i: (i, 0))]
            + _sblocks(0) + _sblocks(0)
            + [
                pl.BlockSpec((3, _D, _D), lambda i: (0, 0, 0)),
                pl.BlockSpec((1, _D), lambda i: (0, 0)),
                pl.BlockSpec((_BLK, 16), lambda i: (i, 0)),
                pl.BlockSpec((_D, _D), lambda i: (0, 0)),
            ]
        ),
        out_specs=[
            pl.BlockSpec((_BLK, _D), lambda i: (i, 0)),
            pl.BlockSpec((_NC, _BLK, _DH), lambda i: (0, i, 0)),
        ],
        out_shape=[
            jax.ShapeDtypeStruct((_N, _D), jnp.float32),
            jax.ShapeDtypeStruct((_NC, _N, _DH), jnp.bfloat16),
        ],
    )(h, sa3d, sa3d, sb3d, sb3d, w, b.reshape(1, _D), dis, m)


def kernel(x, edge, w1, b1, w2, b2):
    n, d = x.shape
    e = edge.shape[1]
    src = edge[0].astype(jnp.int32)
    dst = edge[1].astype(jnp.int32)

    kd = (-(-e // (_NC * _NS * _C)) + 7) // 8 * 8
    pad_d = _NC * _NS * kd * _C - e
    src_deg = jnp.concatenate(
        [src, jnp.full((pad_d,), n, jnp.int32)]).reshape(_NC * _NS * kd, _C)

    kp = (-(-e // (_NS * _C)) + 7) // 8 * 8
    pad_p = _NS * kp * _C - e
    src_p = jnp.concatenate([src, jnp.zeros((pad_p,), jnp.int32)])
    src_fs = jnp.concatenate(
        [src_p, src_p + jnp.int32(n)]).reshape(_NC * _NS * kp, _C)
    dst_fs = jnp.concatenate(
        [dst, jnp.full((pad_p,), n, jnp.int32)]).reshape(_NS * kp, _C)

    zeros_h = jnp.zeros((_NACC, _DH), jnp.float32)
    zeros16 = jnp.zeros((_NACC, 16), jnp.float32)
    ones16 = jnp.ones((_C, 16), jnp.float32)

    sc_deg = _make_sc_deg(kd)
    sc_prop = _make_sc_prop(kp)

    t64 = np.empty((64,), np.int32)
    for u in range(2):
        for k in range(16):
            t64[32 * u + k] = 32 * u + 2 * k
            t64[32 * u + 16 + k] = 32 * u + 2 * k + 1
    t128 = np.concatenate([t64, t64 + 64])
    perm = np.argsort(t128)
    m_np = np.zeros((_D, _D), np.float32)
    m_np[perm, np.arange(_D)] = 1.0
    m = jnp.asarray(m_np)

    deg_parts = sc_deg(src_deg, zeros16, ones16).reshape(_NC, _NACC, 16)
    dis, g0 = _tc_prep(deg_parts, x, m)

    def prop3d(g):
        s = sc_prop(g.reshape(_NC * n, _DH), src_fs, dst_fs, zeros_h)
        return s.reshape(_NC, _NACC, _DH)

    s1 = prop3d(g0)
    g1 = _tc_scaleg(s1, dis, m)
    s2 = prop3d(g1)
    out1, g2 = _tc_dense(x, s1, s2, w1, b1, dis, m, relu=True)
    s3 = prop3d(g2)
    g3 = _tc_scaleg(s3, dis, m)
    s4 = prop3d(g3)
    out, _ = _tc_dense(out1, s3, s4, w2, b2, dis, m, relu=False)
    return out


# speedup vs baseline: 1.2897x; 1.0065x over previous
import functools

import jax
import numpy as np
import jax.numpy as jnp
from jax import lax
from jax.experimental import pallas as pl
from jax.experimental.pallas import tpu as pltpu
from jax.experimental.pallas import tpu_sc as plsc

_NC = 2
_NS = 16
_C = 128

_N = 10000
_D = 128
_DH = _D // _NC
_NACC = 10240
_ZROWS = _NACC // _NS

_BLK = 2000


def _mesh():
    return plsc.VectorSubcoreMesh(core_axis_name="c", subcore_axis_name="s")


def _make_sc_prop(kp):
    @functools.partial(
        pl.kernel,
        out_type=jax.ShapeDtypeStruct((_NC * _NACC, _DH), jnp.float32),
        mesh=_mesh(),
        scratch_types=[
            pltpu.VMEM((kp, _C), jnp.int32),
            pltpu.VMEM((kp, _C), jnp.int32),
            [pltpu.VMEM((_C, _DH), jnp.bfloat16) for _ in range(4)],
            [pltpu.VMEM((_C, _DH), jnp.float32) for _ in range(2)],
            pltpu.VMEM_SHARED((_NACC, _DH), jnp.float32),
            [pltpu.SemaphoreType.DMA for _ in range(4)],
            [pltpu.SemaphoreType.DMA for _ in range(2)],
        ],
        compiler_params=pltpu.CompilerParams(
            use_tc_tiling_on_sc=False, needs_layout_passes=False),
    )
    def sc_prop(tab_hbm, srcs_hbm, dsts_hbm, zeros_hbm, out_hbm,
                src_v, dst_v, rows, sts, acc_sh, sem_g, sem_s):
        cid = lax.axis_index("c")
        sid = lax.axis_index("s")

        pltpu.sync_copy(zeros_hbm.at[pl.ds(sid * _ZROWS, _ZROWS)],
                        acc_sh.at[pl.ds(sid * _ZROWS, _ZROWS)])
        pltpu.sync_copy(srcs_hbm.at[pl.ds((cid * _NS + sid) * kp, kp)], src_v)
        pltpu.sync_copy(dsts_hbm.at[pl.ds(sid * kp, kp)], dst_v)
        plsc.subcore_barrier()

        def fire_gather(grp, k):
            pltpu.async_copy(tab_hbm.at[src_v.at[grp]], rows[k], sem_g[k])

        def drain_gather(k):
            pltpu.make_async_copy(tab_hbm.at[src_v.at[0]], rows[k],
                                  sem_g[k]).wait()

        def fire_scatter(grp, j):
            pltpu.async_copy(sts[j], acc_sh.at[dst_v.at[grp]], sem_s[j],
                             add=True)

        def drain_scatter(j):
            pltpu.make_async_copy(sts[j], acc_sh.at[dst_v.at[0]],
                                  sem_s[j]).wait()

        def convert(k, j):
            buf, stb = rows[k], sts[j]
            def body(r, carry):
                for u in range(_DH // 32):
                    v = buf[r, pl.ds(32 * u, 32)]
                    a, b = plsc.unpack(v, format=plsc.PackFormat.INTERLEAVED)
                    stb[r, pl.ds(32 * u, 16)] = a
                    stb[r, pl.ds(32 * u + 16, 16)] = b
                return carry
            lax.fori_loop(0, _C, body, 0, unroll=4)

        def phase(grp, k, nxt, drain_st):
            j = k % 2
            drain_gather(k)
            if drain_st:
                drain_scatter(j)
            convert(k, j)
            fire_scatter(grp, j)
            fire_gather(nxt, k)

        for k in range(4):
            fire_gather(k, k)
        phase(0, 0, 4, drain_st=False)
        phase(1, 1, 5, drain_st=False)
        phase(2, 2, 6, drain_st=True)
        phase(3, 3, 7, drain_st=True)

        def pipe(i, carry):
            t = 4 * i
            for k in range(4):
                phase(t + k, k, lax.rem(t + k + 4, kp), drain_st=True)
            return carry

        lax.fori_loop(1, kp // 4, pipe, 0)
        drain_scatter(0)
        drain_scatter(1)
        for k in range(4):
            drain_gather(k)
        plsc.subcore_barrier()

        pltpu.sync_copy(
            acc_sh.at[pl.ds(sid * _ZROWS, _ZROWS)],
            out_hbm.at[pl.ds(cid * _NACC + sid * _ZROWS, _ZROWS)])

    return sc_prop


def _make_sc_deg(kd):
    @functools.partial(
        pl.kernel,
        out_type=jax.ShapeDtypeStruct((_NC * _NACC, 16), jnp.float32),
        mesh=_mesh(),
        scratch_types=[
            pltpu.VMEM((kd, _C), jnp.int32),
            pltpu.VMEM((_C, 16), jnp.float32),
            pltpu.VMEM_SHARED((_NACC, 16), jnp.float32),
        ],
        compiler_params=pltpu.CompilerParams(
            use_tc_tiling_on_sc=False, needs_layout_passes=False),
    )
    def sc_deg(srcs_hbm, zeros_hbm, ones_hbm, out_hbm,
               src_v, ones_v, acc_sh):
        cid = lax.axis_index("c")
        sid = lax.axis_index("s")
        wid = sid * _NC + cid

        pltpu.sync_copy(zeros_hbm.at[pl.ds(sid * _ZROWS, _ZROWS)],
                        acc_sh.at[pl.ds(sid * _ZROWS, _ZROWS)])
        pltpu.sync_copy(ones_hbm, ones_v)
        pltpu.sync_copy(srcs_hbm.at[pl.ds(wid * kd, kd)], src_v)
        plsc.subcore_barrier()

        def chunk(j, carry):
            pltpu.sync_copy(ones_v, acc_sh.at[src_v.at[j]], add=True)
            return carry

        lax.fori_loop(0, kd, chunk, 0)
        plsc.subcore_barrier()

        pltpu.sync_copy(
            acc_sh.at[pl.ds(sid * _ZROWS, _ZROWS)],
            out_hbm.at[pl.ds(cid * _NACC + sid * _ZROWS, _ZROWS)])

    return sc_deg


def _sblocks(i):
    del i
    return [
        pl.BlockSpec((1, _BLK, _DH), lambda i: (0, i, 0)),
        pl.BlockSpec((1, _BLK, _DH), lambda i: (1, i, 0)),
    ]


def _prep_body(dp_ref, x_ref, m_ref, dis_ref, g0_ref):
    dp = dp_ref[...]
    deg = dp[0] + dp[1]
    dis = jnp.where(deg > 0, lax.rsqrt(jnp.where(deg > 0, deg, 1.0)), 0.0)
    dis_ref[...] = dis
    dis_c = dis[:, 0:1]
    t = jnp.dot(dis_c * x_ref[...], m_ref[...],
                preferred_element_type=jnp.float32).astype(jnp.bfloat16)
    g0_ref[...] = jnp.stack([t[:, :_DH], t[:, _DH:]])


def _tc_prep(deg_parts, x, m):
    grid = _N // _BLK
    return pl.pallas_call(
        _prep_body,
        grid=(grid,),
        in_specs=[
            pl.BlockSpec((_NC, _BLK, 16), lambda i: (0, i, 0)),
            pl.BlockSpec((_BLK, _D), lambda i: (i, 0)),
            pl.BlockSpec((_D, _D), lambda i: (0, 0)),
        ],
        out_specs=[
            pl.BlockSpec((_BLK, 16), lambda i: (i, 0)),
            pl.BlockSpec((_NC, _BLK, _DH), lambda i: (0, i, 0)),
        ],
        out_shape=[
            jax.ShapeDtypeStruct((_N, 16), jnp.float32),
            jax.ShapeDtypeStruct((_NC, _N, _DH), jnp.bfloat16),
        ],
    )(deg_parts, x, m)


def _scaleg_body(s0_ref, s1_ref, dis_ref, m_ref, g_ref):
    nd2 = -jnp.square(dis_ref[...][:, 0:1])
    full = jnp.concatenate([nd2 * s0_ref[0], nd2 * s1_ref[0]], axis=1)
    t = jnp.dot(full, m_ref[...],
                preferred_element_type=jnp.float32).astype(jnp.bfloat16)
    g_ref[...] = jnp.stack([t[:, :_DH], t[:, _DH:]])


def _tc_scaleg(s3d, dis, m):
    grid = _N // _BLK
    return pl.pallas_call(
        _scaleg_body,
        grid=(grid,),
        in_specs=_sblocks(0) + [
            pl.BlockSpec((_BLK, 16), lambda i: (i, 0)),
            pl.BlockSpec((_D, _D), lambda i: (0, 0)),
        ],
        out_specs=pl.BlockSpec((_NC, _BLK, _DH), lambda i: (0, i, 0)),
        out_shape=jax.ShapeDtypeStruct((_NC, _N, _DH), jnp.bfloat16),
    )(s3d, s3d, dis, m)


def _dense_body(relu, h_ref, sa0_ref, sa1_ref, sb0_ref, sb1_ref,
                w_ref, b_ref, dis_ref, m_ref, out_ref, g_ref):
    h = h_ref[...]
    nd = -dis_ref[...][:, 0:1]
    p1 = jnp.concatenate([nd * sa0_ref[0], nd * sa1_ref[0]], axis=1)
    p2 = jnp.concatenate([nd * sb0_ref[0], nd * sb1_ref[0]], axis=1)
    acc = jnp.dot(h, w_ref[0], preferred_element_type=jnp.float32)
    acc += jnp.dot(p1, w_ref[1], preferred_element_type=jnp.float32)
    acc += jnp.dot(2.0 * p2 - h, w_ref[2],
                   preferred_element_type=jnp.float32)
    acc += b_ref[...]
    if relu:
        acc = jnp.maximum(acc, 0.0)
    out_ref[...] = acc
    dis_c = dis_ref[...][:, 0:1]
    t = jnp.dot(dis_c * acc, m_ref[...],
                preferred_element_type=jnp.float32).astype(jnp.bfloat16)
    g_ref[...] = jnp.stack([t[:, :_DH], t[:, _DH:]])


def _tc_dense(h, sa3d, sb3d, w, b, dis, m, relu):
    grid = _N // _BLK
    return pl.pallas_call(
        functools.partial(_dense_body, relu),
        grid=(grid,),
        in_specs=(
            [pl.BlockSpec((_BLK, _D), lambda i: (i, 0))]
            + _sblocks(0) + _sblocks(0)
            + [
                pl.BlockSpec((3, _D, _D), lambda i: (0, 0, 0)),
                pl.BlockSpec((1, _D), lambda i: (0, 0)),
                pl.BlockSpec((_BLK, 16), lambda i: (i, 0)),
                pl.BlockSpec((_D, _D), lambda i: (0, 0)),
            ]
        ),
        out_specs=[
            pl.BlockSpec((_BLK, _D), lambda i: (i, 0)),
            pl.BlockSpec((_NC, _BLK, _DH), lambda i: (0, i, 0)),
        ],
        out_shape=[
            jax.ShapeDtypeStruct((_N, _D), jnp.float32),
            jax.ShapeDtypeStruct((_NC, _N, _DH), jnp.bfloat16),
        ],
    )(h, sa3d, sa3d, sb3d, sb3d, w, b.reshape(1, _D), dis, m)


def kernel(x, edge, w1, b1, w2, b2):
    n, d = x.shape
    e = edge.shape[1]
    src = edge[0].astype(jnp.int32)
    dst = edge[1].astype(jnp.int32)

    kd = (-(-e // (_NC * _NS * _C)) + 7) // 8 * 8
    pad_d = _NC * _NS * kd * _C - e
    src_deg = jnp.concatenate(
        [src, jnp.full((pad_d,), n, jnp.int32)]).reshape(_NC * _NS * kd, _C)

    kp = (-(-e // (_NS * _C)) + 7) // 8 * 8
    pad_p = _NS * kp * _C - e
    src_p = jnp.concatenate([src, jnp.zeros((pad_p,), jnp.int32)])
    src_fs = jnp.concatenate(
        [src_p, src_p + jnp.int32(n)]).reshape(_NC * _NS * kp, _C)
    dst_fs = jnp.concatenate(
        [dst, jnp.full((pad_p,), n, jnp.int32)]).reshape(_NS * kp, _C)

    zeros_h = jnp.zeros((_NACC, _DH), jnp.float32)
    zeros16 = jnp.zeros((_NACC, 16), jnp.float32)
    ones16 = jnp.ones((_C, 16), jnp.float32)

    sc_deg = _make_sc_deg(kd)
    sc_prop = _make_sc_prop(kp)

    t64 = np.empty((64,), np.int32)
    for u in range(2):
        for k in range(16):
            t64[32 * u + k] = 32 * u + 2 * k
            t64[32 * u + 16 + k] = 32 * u + 2 * k + 1
    t128 = np.concatenate([t64, t64 + 64])
    perm = np.argsort(t128)
    m_np = np.zeros((_D, _D), np.float32)
    m_np[perm, np.arange(_D)] = 1.0
    m = jnp.asarray(m_np)

    deg_parts = sc_deg(src_deg, zeros16, ones16).reshape(_NC, _NACC, 16)
    dis, g0 = _tc_prep(deg_parts, x, m)

    def prop3d(g):
        s = sc_prop(g.reshape(_NC * n, _DH), src_fs, dst_fs, zeros_h)
        return s.reshape(_NC, _NACC, _DH)

    s1 = prop3d(g0)
    g1 = _tc_scaleg(s1, dis, m)
    s2 = prop3d(g1)
    out1, g2 = _tc_dense(x, s1, s2, w1, b1, dis, m, relu=True)
    s3 = prop3d(g2)
    g3 = _tc_scaleg(s3, dis, m)
    s4 = prop3d(g3)
    out, _ = _tc_dense(out1, s3, s4, w2, b2, dis, m, relu=False)
    return out


# speedup vs baseline: 1.3133x; 1.0183x over previous
import functools

import jax
import numpy as np
import jax.numpy as jnp
from jax import lax
from jax.experimental import pallas as pl
from jax.experimental.pallas import tpu as pltpu
from jax.experimental.pallas import tpu_sc as plsc

_NC = 2
_NS = 16
_C = 128

_N = 10000
_D = 128
_DH = _D // _NC
_NACC = 10240
_ZROWS = _NACC // _NS

_BLK = 2000


def _mesh():
    return plsc.VectorSubcoreMesh(core_axis_name="c", subcore_axis_name="s")


def _make_sc_prop(kp, emit_table):
    s_type = jax.ShapeDtypeStruct((_NC * _NACC, _DH), jnp.float32)
    t_type = jax.ShapeDtypeStruct((_NC * _NACC, _DH), jnp.bfloat16)

    @functools.partial(
        pl.kernel,
        out_type=(s_type, t_type) if emit_table else s_type,
        mesh=_mesh(),
        scratch_types=[
            pltpu.VMEM((kp, _C), jnp.int32),
            pltpu.VMEM((kp, _C), jnp.int32),
            [pltpu.VMEM((_C, _DH), jnp.bfloat16) for _ in range(4)],
            [pltpu.VMEM((_C, _DH), jnp.float32) for _ in range(2)],
            pltpu.VMEM_SHARED((_NACC, _DH), jnp.float32),
            [pltpu.SemaphoreType.DMA for _ in range(4)],
            [pltpu.SemaphoreType.DMA for _ in range(2)],
            pltpu.VMEM((_ZROWS,), jnp.float32),
        ],
        compiler_params=pltpu.CompilerParams(
            use_tc_tiling_on_sc=False, needs_layout_passes=False),
    )
    def sc_prop(tab_hbm, srcs_hbm, dsts_hbm, zeros_hbm, nd2_hbm, *rest):
        if emit_table:
            (s_hbm, t_hbm), refs = rest[:2], rest[2:]
        else:
            s_hbm, refs = rest[0], rest[1:]
        src_v, dst_v, rows, sts, acc_sh, sem_g, sem_s, nd2_v = refs
        cid = lax.axis_index("c")
        sid = lax.axis_index("s")

        pltpu.sync_copy(zeros_hbm.at[pl.ds(sid * _ZROWS, _ZROWS)],
                        acc_sh.at[pl.ds(sid * _ZROWS, _ZROWS)])
        pltpu.sync_copy(srcs_hbm.at[pl.ds((cid * _NS + sid) * kp, kp)], src_v)
        pltpu.sync_copy(dsts_hbm.at[pl.ds(sid * kp, kp)], dst_v)
        plsc.subcore_barrier()

        def fire_gather(grp, k):
            pltpu.async_copy(tab_hbm.at[src_v.at[grp]], rows[k], sem_g[k])

        def drain_gather(k):
            pltpu.make_async_copy(tab_hbm.at[src_v.at[0]], rows[k],
                                  sem_g[k]).wait()

        def fire_scatter(grp, j):
            pltpu.async_copy(sts[j], acc_sh.at[dst_v.at[grp]], sem_s[j],
                             add=True)

        def drain_scatter(j):
            pltpu.make_async_copy(sts[j], acc_sh.at[dst_v.at[0]],
                                  sem_s[j]).wait()

        def convert(k, j):
            buf, stb = rows[k], sts[j]
            def body(r, carry):
                for u in range(_DH // 32):
                    v = buf[r, pl.ds(32 * u, 32)]
                    a, b = plsc.unpack(v, format=plsc.PackFormat.INTERLEAVED)
                    stb[r, pl.ds(32 * u, 16)] = a
                    stb[r, pl.ds(32 * u + 16, 16)] = b
                return carry
            lax.fori_loop(0, _C, body, 0, unroll=4)

        def phase(grp, k, nxt, drain_st):
            j = k % 2
            drain_gather(k)
            if drain_st:
                drain_scatter(j)
            convert(k, j)
            fire_scatter(grp, j)
            fire_gather(nxt, k)

        for k in range(4):
            fire_gather(k, k)
        phase(0, 0, 4, drain_st=False)
        phase(1, 1, 5, drain_st=False)
        phase(2, 2, 6, drain_st=True)
        phase(3, 3, 7, drain_st=True)

        def pipe(i, carry):
            t = 4 * i
            for k in range(4):
                phase(t + k, k, lax.rem(t + k + 4, kp), drain_st=True)
            return carry

        lax.fori_loop(1, kp // 4, pipe, 0)
        drain_scatter(0)
        drain_scatter(1)
        for k in range(4):
            drain_gather(k)
        plsc.subcore_barrier()

        pltpu.sync_copy(
            acc_sh.at[pl.ds(sid * _ZROWS, _ZROWS)],
            s_hbm.at[pl.ds(cid * _NACC + sid * _ZROWS, _ZROWS)])

        if emit_table:
            pltpu.sync_copy(nd2_hbm.at[pl.ds(sid * _ZROWS, _ZROWS)], nd2_v)
            nch = _ZROWS // _C
            for cc in range(nch):
                base = sid * _ZROWS + cc * _C
                pltpu.sync_copy(acc_sh.at[pl.ds(base, _C)], sts[0])

                def trow16(g16, carry):
                    dvec = nd2_v[pl.ds(cc * _C + g16 * 16, 16)]
                    for r16 in range(16):
                        r = g16 * 16 + r16
                        d = dvec[r16]
                        for u in range(_DH // 32):
                            a = d * sts[0][r, pl.ds(32 * u, 16)]
                            b = d * sts[0][r, pl.ds(32 * u + 16, 16)]
                            rows[0][r, pl.ds(32 * u, 32)] = plsc.pack(
                                a, b, format=plsc.PackFormat.INTERLEAVED)
                    return carry

                lax.fori_loop(0, _C // 16, trow16, 0)
                pltpu.sync_copy(
                    rows[0], t_hbm.at[pl.ds(cid * _NACC + base, _C)])

    return sc_prop


def _make_sc_deg(kd):
    @functools.partial(
        pl.kernel,
        out_type=jax.ShapeDtypeStruct((_NC * _NACC, 16), jnp.float32),
        mesh=_mesh(),
        scratch_types=[
            pltpu.VMEM((kd, _C), jnp.int32),
            pltpu.VMEM((_C, 16), jnp.float32),
            pltpu.VMEM_SHARED((_NACC, 16), jnp.float32),
        ],
        compiler_params=pltpu.CompilerParams(
            use_tc_tiling_on_sc=False, needs_layout_passes=False),
    )
    def sc_deg(srcs_hbm, zeros_hbm, ones_hbm, out_hbm,
               src_v, ones_v, acc_sh):
        cid = lax.axis_index("c")
        sid = lax.axis_index("s")
        wid = sid * _NC + cid

        pltpu.sync_copy(zeros_hbm.at[pl.ds(sid * _ZROWS, _ZROWS)],
                        acc_sh.at[pl.ds(sid * _ZROWS, _ZROWS)])
        pltpu.sync_copy(ones_hbm, ones_v)
        pltpu.sync_copy(srcs_hbm.at[pl.ds(wid * kd, kd)], src_v)
        plsc.subcore_barrier()

        def chunk(j, carry):
            pltpu.sync_copy(ones_v, acc_sh.at[src_v.at[j]], add=True)
            return carry

        lax.fori_loop(0, kd, chunk, 0)
        plsc.subcore_barrier()

        pltpu.sync_copy(
            acc_sh.at[pl.ds(sid * _ZROWS, _ZROWS)],
            out_hbm.at[pl.ds(cid * _NACC + sid * _ZROWS, _ZROWS)])

    return sc_deg


def _sblocks(i):
    del i
    return [
        pl.BlockSpec((1, _BLK, _DH), lambda i: (0, i, 0)),
        pl.BlockSpec((1, _BLK, _DH), lambda i: (1, i, 0)),
    ]


def _prep_body(dp_ref, x_ref, m_ref, dis_ref, nd2_ref, g0_ref):
    dp = dp_ref[...]
    deg = dp[0] + dp[1]
    dis = jnp.where(deg > 0, lax.rsqrt(jnp.where(deg > 0, deg, 1.0)), 0.0)
    dis_ref[...] = dis
    nd2_ref[...] = -jnp.square(dis)
    dis_c = dis[:, 0:1]
    t = jnp.dot(dis_c * x_ref[...], m_ref[...],
                preferred_element_type=jnp.float32).astype(jnp.bfloat16)
    g0_ref[...] = jnp.stack([t[:, :_DH], t[:, _DH:]])


def _tc_prep(deg_parts, x, m):
    grid = _N // _BLK
    return pl.pallas_call(
        _prep_body,
        grid=(grid,),
        in_specs=[
            pl.BlockSpec((_NC, _BLK, 16), lambda i: (0, i, 0)),
            pl.BlockSpec((_BLK, _D), lambda i: (i, 0)),
            pl.BlockSpec((_D, _D), lambda i: (0, 0)),
        ],
        out_specs=[
            pl.BlockSpec((_BLK, 16), lambda i: (i, 0)),
            pl.BlockSpec((_BLK, 16), lambda i: (i, 0)),
            pl.BlockSpec((_NC, _BLK, _DH), lambda i: (0, i, 0)),
        ],
        out_shape=[
            jax.ShapeDtypeStruct((_N, 16), jnp.float32),
            jax.ShapeDtypeStruct((_N, 16), jnp.float32),
            jax.ShapeDtypeStruct((_NC, _N, _DH), jnp.bfloat16),
        ],
    )(deg_parts, x, m)


def _dense_body(relu, h_ref, sa0_ref, sa1_ref, sb0_ref, sb1_ref,
                w_ref, b_ref, dis_ref, m_ref, out_ref, g_ref):
    h = h_ref[...]
    nd = -dis_ref[...][:, 0:1]
    p1 = jnp.concatenate([nd * sa0_ref[0], nd * sa1_ref[0]], axis=1)
    p2 = jnp.concatenate([nd * sb0_ref[0], nd * sb1_ref[0]], axis=1)
    acc = jnp.dot(h, w_ref[0], preferred_element_type=jnp.float32)
    acc += jnp.dot(p1, w_ref[1], preferred_element_type=jnp.float32)
    acc += jnp.dot(2.0 * p2 - h, w_ref[2],
                   preferred_element_type=jnp.float32)
    acc += b_ref[...]
    if relu:
        acc = jnp.maximum(acc, 0.0)
    out_ref[...] = acc
    dis_c = dis_ref[...][:, 0:1]
    t = jnp.dot(dis_c * acc, m_ref[...],
                preferred_element_type=jnp.float32).astype(jnp.bfloat16)
    g_ref[...] = jnp.stack([t[:, :_DH], t[:, _DH:]])


def _tc_dense(h, sa3d, sb3d, w, b, dis, m, relu):
    grid = _N // _BLK
    return pl.pallas_call(
        functools.partial(_dense_body, relu),
        grid=(grid,),
        in_specs=(
            [pl.BlockSpec((_BLK, _D), lambda i: (i, 0))]
            + _sblocks(0) + _sblocks(0)
            + [
                pl.BlockSpec((3, _D, _D), lambda i: (0, 0, 0)),
                pl.BlockSpec((1, _D), lambda i: (0, 0)),
                pl.BlockSpec((_BLK, 16), lambda i: (i, 0)),
                pl.BlockSpec((_D, _D), lambda i: (0, 0)),
            ]
        ),
        out_specs=[
            pl.BlockSpec((_BLK, _D), lambda i: (i, 0)),
            pl.BlockSpec((_NC, _BLK, _DH), lambda i: (0, i, 0)),
        ],
        out_shape=[
            jax.ShapeDtypeStruct((_N, _D), jnp.float32),
            jax.ShapeDtypeStruct((_NC, _N, _DH), jnp.bfloat16),
        ],
    )(h, sa3d, sa3d, sb3d, sb3d, w, b.reshape(1, _D), dis, m)


def kernel(x, edge, w1, b1, w2, b2):
    n, d = x.shape
    e = edge.shape[1]
    src = edge[0].astype(jnp.int32)
    dst = edge[1].astype(jnp.int32)

    kd = (-(-e // (_NC * _NS * _C)) + 7) // 8 * 8
    pad_d = _NC * _NS * kd * _C - e
    src_deg = jnp.concatenate(
        [src, jnp.full((pad_d,), n, jnp.int32)]).reshape(_NC * _NS * kd, _C)

    kp = (-(-e // (_NS * _C)) + 7) // 8 * 8
    pad_p = _NS * kp * _C - e
    src_p = jnp.concatenate([src, jnp.zeros((pad_p,), jnp.int32)])
    src_fs = jnp.concatenate(
        [src_p, src_p + jnp.int32(n)]).reshape(_NC * _NS * kp, _C)
    src_fs_na = jnp.concatenate(
        [src_p, src_p + jnp.int32(_NACC)]).reshape(_NC * _NS * kp, _C)
    dst_fs = jnp.concatenate(
        [dst, jnp.full((pad_p,), n, jnp.int32)]).reshape(_NS * kp, _C)

    zeros_h = jnp.zeros((_NACC, _DH), jnp.float32)
    zeros16 = jnp.zeros((_NACC, 16), jnp.float32)
    ones16 = jnp.ones((_C, 16), jnp.float32)

    sc_deg = _make_sc_deg(kd)
    sc_prop_a = _make_sc_prop(kp, emit_table=True)
    sc_prop_b = _make_sc_prop(kp, emit_table=False)

    t64 = np.empty((64,), np.int32)
    for u in range(2):
        for k in range(16):
            t64[32 * u + k] = 32 * u + 2 * k
            t64[32 * u + 16 + k] = 32 * u + 2 * k + 1
    t128 = np.concatenate([t64, t64 + 64])
    perm = np.argsort(t128)
    m_np = np.zeros((_D, _D), np.float32)
    m_np[perm, np.arange(_D)] = 1.0
    m = jnp.asarray(m_np)

    deg_parts = sc_deg(src_deg, zeros16, ones16).reshape(_NC, _NACC, 16)
    dis, nd2, g0 = _tc_prep(deg_parts, x, m)
    nd2_full = jnp.pad(nd2[:, 0], (0, _NACC - n))

    def prop_pair(g):
        s_i, t_i = sc_prop_a(g.reshape(_NC * n, _DH), src_fs, dst_fs,
                             zeros_h, nd2_full)
        s_ii = sc_prop_b(t_i, src_fs_na, dst_fs, zeros_h, nd2_full)
        return s_i.reshape(_NC, _NACC, _DH), s_ii.reshape(_NC, _NACC, _DH)

    s1, s2 = prop_pair(g0)
    out1, g2 = _tc_dense(x, s1, s2, w1, b1, dis, m, relu=True)
    s3, s4 = prop_pair(g2)
    out, _ = _tc_dense(out1, s3, s4, w2, b2, dis, m, relu=False)
    return out
